# baseline (speedup 1.0000x reference)
import functools
import math

import jax
import jax.numpy as jnp
from jax.experimental import pallas as pl
from jax.experimental.pallas import tpu as pltpu

_TINY = 1.1754943508222875e-38
_KB = 512


def _rotl(x, r):
    return (x << jnp.uint32(r)) | (x >> jnp.uint32(32 - r))


def _threefry2x32_bits(idx):
    k0 = jnp.uint32(0)
    k1 = jnp.uint32(42)
    ks2 = k0 ^ k1 ^ jnp.uint32(0x1BD11BDA)
    ks = (k0, k1, ks2)
    rot = ((13, 15, 26, 6), (17, 29, 16, 24))
    x0 = jnp.zeros_like(idx) + k0
    x1 = idx + k1
    for i in range(5):
        for r in rot[i % 2]:
            x0 = x0 + x1
            x1 = _rotl(x1, r)
            x1 = x1 ^ x0
        x0 = x0 + ks[(i + 1) % 3]
        x1 = x1 + ks[(i + 2) % 3] + jnp.uint32(i + 1)
    return x0 ^ x1


def _body(xs_ref, m_ref, cov_ref, out_ref, best_ref, bidx_ref, *, K, B, D, nb):
    i = pl.program_id(0)

    @pl.when(i == 0)
    def _init():
        best_ref[...] = jnp.full((B, 1), -jnp.inf, jnp.float32)
        bidx_ref[...] = jnp.zeros((B, 1), jnp.int32)

    xs = xs_ref[...]
    m = m_ref[...]
    var = cov_ref[0, 0]

    dot = jax.lax.dot_general(xs, m, (((1,), (1,)), ((), ())),
                              preferred_element_type=jnp.float32)
    xsq = jnp.sum(xs * xs, axis=1, keepdims=True)
    msq = jnp.sum(m * m, axis=1)[None, :]
    sq = xsq + msq - 2.0 * dot
    logp = -0.5 * sq / var - 0.5 * D * jnp.log(2.0 * jnp.pi * var)

    row = jax.lax.broadcasted_iota(jnp.uint32, (B, _KB), 0)
    col = jax.lax.broadcasted_iota(jnp.uint32, (B, _KB), 1)
    idx = row * jnp.uint32(K) + (col + (i * _KB).astype(jnp.uint32))
    bits = _threefry2x32_bits(idx)
    fbits = (bits >> jnp.uint32(9)) | jnp.uint32(0x3F800000)
    u = jax.lax.bitcast_convert_type(fbits, jnp.float32) - 1.0
    u = jnp.maximum(_TINY, u + _TINY)
    g = -jnp.log(-jnp.log(u))

    score = g + logp
    kglob = i * _KB + jax.lax.broadcasted_iota(jnp.int32, (B, _KB), 1)
    score = jnp.where(kglob < K, score, -jnp.inf)

    bm = jnp.max(score, axis=1, keepdims=True)
    cand = jnp.where(score == bm, kglob, jnp.int32(2**31 - 1))
    bi = jnp.min(cand, axis=1, keepdims=True)

    upd = bm > best_ref[...]
    best_ref[...] = jnp.where(upd, bm, best_ref[...])
    bidx_ref[...] = jnp.where(upd, bi, bidx_ref[...])

    @pl.when(i == nb - 1)
    def _emit():
        out_ref[...] = bidx_ref[...]


def kernel(xs, means, cov):
    B, D = xs.shape
    K = means.shape[0]
    nb = math.ceil(K / _KB)
    k_pad = nb * _KB
    means_p = jnp.pad(means, ((0, k_pad - K), (0, 0)))
    cov2 = cov.reshape(1, 1)

    out = pl.pallas_call(
        functools.partial(_body, K=K, B=B, D=D, nb=nb),
        grid=(nb,),
        in_specs=[
            pl.BlockSpec((B, D), lambda i: (0, 0)),
            pl.BlockSpec((_KB, D), lambda i: (i, 0)),
            pl.BlockSpec((1, 1), lambda i: (0, 0)),
        ],
        out_specs=pl.BlockSpec((B, 1), lambda i: (0, 0)),
        out_shape=jax.ShapeDtypeStruct((B, 1), jnp.int32),
        scratch_shapes=[
            pltpu.VMEM((B, 1), jnp.float32),
            pltpu.VMEM((B, 1), jnp.int32),
        ],
    )(xs, means_p, cov2)
    return out[:, 0]


# speedup vs baseline: 1.0730x; 1.0730x over previous
import functools
import math

import jax
import jax.numpy as jnp
from jax.experimental import pallas as pl
from jax.experimental.pallas import tpu as pltpu

_TINY = 1.1754943508222875e-38
_KB = 1024
_PAD_MEAN = 1.0e18


def _rotl(x, r):
    return (x << jnp.uint32(r)) | (x >> jnp.uint32(32 - r))


def _threefry2x32_bits(idx):
    k0 = jnp.uint32(0)
    k1 = jnp.uint32(42)
    ks2 = k0 ^ k1 ^ jnp.uint32(0x1BD11BDA)
    ks = (k0, k1, ks2)
    rot = ((13, 15, 26, 6), (17, 29, 16, 24))
    x0 = jnp.zeros_like(idx) + k0
    x1 = idx + k1
    for i in range(5):
        for r in rot[i % 2]:
            x0 = x0 + x1
            x1 = _rotl(x1, r)
            x1 = x1 ^ x0
        x0 = x0 + ks[(i + 1) % 3]
        x1 = x1 + ks[(i + 2) % 3] + jnp.uint32(i + 1)
    return x0 ^ x1


def _body(xs_ref, mt_ref, cov_ref, out_ref, best_ref, bidx_ref, *, K, B, D, nb):
    i = pl.program_id(0)

    @pl.when(i == 0)
    def _init():
        best_ref[...] = jnp.full((B, 1), -jnp.inf, jnp.float32)
        bidx_ref[...] = jnp.zeros((B, 1), jnp.int32)

    xs = xs_ref[...]
    mt = mt_ref[...]
    var = cov_ref[0, 0]
    scale = -0.5 / var
    cst = -0.5 * D * jnp.log(2.0 * jnp.pi * var)

    dot = jax.lax.dot_general(xs, mt, (((1,), (0,)), ((), ())),
                              preferred_element_type=jnp.float32)
    xsq = jnp.sum(xs * xs, axis=1, keepdims=True)
    msq = jnp.sum(mt * mt, axis=0, keepdims=True)
    sq = xsq + msq - 2.0 * dot
    logp = sq * scale + cst

    row = jax.lax.broadcasted_iota(jnp.uint32, (B, _KB), 0)
    col = jax.lax.broadcasted_iota(jnp.uint32, (B, _KB), 1)
    idx = row * jnp.uint32(K) + (col + (i * _KB).astype(jnp.uint32))
    bits = _threefry2x32_bits(idx)
    fbits = (bits >> jnp.uint32(9)) | jnp.uint32(0x3F800000)
    u = jax.lax.bitcast_convert_type(fbits, jnp.float32) - 1.0
    u = jnp.maximum(_TINY, u + _TINY)
    g = -jnp.log(-jnp.log(u))

    score = g + logp

    kglob = i * _KB + jax.lax.broadcasted_iota(jnp.int32, (B, _KB), 1)
    bm = jnp.max(score, axis=1, keepdims=True)
    cand = jnp.where(score == bm, kglob, jnp.int32(2**31 - 1))
    bi = jnp.min(cand, axis=1, keepdims=True)

    upd = bm > best_ref[...]
    best_ref[...] = jnp.where(upd, bm, best_ref[...])
    bidx_ref[...] = jnp.where(upd, bi, bidx_ref[...])

    @pl.when(i == nb - 1)
    def _emit():
        out_ref[...] = bidx_ref[...]


def kernel(xs, means, cov):
    B, D = xs.shape
    K = means.shape[0]
    nb = math.ceil(K / _KB)
    k_pad = nb * _KB
    meansT = jnp.pad(means.T, ((0, 0), (0, k_pad - K)),
                     constant_values=_PAD_MEAN)
    cov2 = cov.reshape(1, 1)

    out = pl.pallas_call(
        functools.partial(_body, K=K, B=B, D=D, nb=nb),
        grid=(nb,),
        in_specs=[
            pl.BlockSpec((B, D), lambda i: (0, 0)),
            pl.BlockSpec((D, _KB), lambda i: (0, i)),
            pl.BlockSpec((1, 1), lambda i: (0, 0)),
        ],
        out_specs=pl.BlockSpec((B, 1), lambda i: (0, 0)),
        out_shape=jax.ShapeDtypeStruct((B, 1), jnp.int32),
        scratch_shapes=[
            pltpu.VMEM((B, 1), jnp.float32),
            pltpu.VMEM((B, 1), jnp.int32),
        ],
    )(xs, meansT, cov2)
    return out[:, 0]


# speedup vs baseline: 1.1011x; 1.0262x over previous
import functools
import math

import jax
import jax.numpy as jnp
from jax.experimental import pallas as pl
from jax.experimental.pallas import tpu as pltpu

_TINY = 1.1754943508222875e-38
_KB = 1024
_PAD_MEAN = 1.0e18


def _rotl(x, r):
    return (x << jnp.uint32(r)) | (x >> jnp.uint32(32 - r))


def _threefry2x32_bits(idx):
    k0 = jnp.uint32(0)
    k1 = jnp.uint32(42)
    ks2 = k0 ^ k1 ^ jnp.uint32(0x1BD11BDA)
    ks = (k0, k1, ks2)
    rot = ((13, 15, 26, 6), (17, 29, 16, 24))
    x1 = idx + k1
    x0 = x1
    x1 = _rotl(x1, rot[0][0])
    x1 = x1 ^ x0
    first = True
    for i in range(5):
        for r in rot[i % 2]:
            if first:
                first = False
                continue
            x0 = x0 + x1
            x1 = _rotl(x1, r)
            x1 = x1 ^ x0
        x0 = x0 + ks[(i + 1) % 3]
        x1 = x1 + ks[(i + 2) % 3] + jnp.uint32(i + 1)
    return x0 ^ x1


def _body(xs_ref, mt_ref, cov_ref, out_ref, best_ref, bidx_ref, *, K, B, D, nb):
    i = pl.program_id(0)

    @pl.when(i == 0)
    def _init():
        best_ref[...] = jnp.full((B, 1), -jnp.inf, jnp.float32)
        bidx_ref[...] = jnp.zeros((B, 1), jnp.int32)

    xs = xs_ref[...]
    mt = mt_ref[...]
    var = cov_ref[0, 0]

    dot = jax.lax.dot_general(xs * (1.0 / var), mt, (((1,), (0,)), ((), ())),
                              preferred_element_type=jnp.float32)
    mc = jnp.sum(mt * mt, axis=0, keepdims=True) * (-0.5 / var)

    row = jax.lax.broadcasted_iota(jnp.uint32, (B, _KB), 0)
    col = jax.lax.broadcasted_iota(jnp.uint32, (B, _KB), 1)
    idx = row * jnp.uint32(K) + (col + (i * _KB).astype(jnp.uint32))
    bits = _threefry2x32_bits(idx)
    fbits = (bits >> jnp.uint32(9)) | jnp.uint32(0x3F800000)
    u = jax.lax.bitcast_convert_type(fbits, jnp.float32) - 1.0
    u = jnp.maximum(_TINY, u + _TINY)
    g = -jnp.log(-jnp.log(u))

    score = (g + dot) + mc

    kglob = i * _KB + jax.lax.broadcasted_iota(jnp.int32, (B, _KB), 1)
    bm = jnp.max(score, axis=1, keepdims=True)
    cand = jnp.where(score == bm, kglob, jnp.int32(2**31 - 1))
    bi = jnp.min(cand, axis=1, keepdims=True)

    upd = bm > best_ref[...]
    best_ref[...] = jnp.where(upd, bm, best_ref[...])
    bidx_ref[...] = jnp.where(upd, bi, bidx_ref[...])

    @pl.when(i == nb - 1)
    def _emit():
        out_ref[...] = bidx_ref[...]


def kernel(xs, means, cov):
    B, D = xs.shape
    K = means.shape[0]
    nb = math.ceil(K / _KB)
    k_pad = nb * _KB
    meansT = jnp.pad(means.T, ((0, 0), (0, k_pad - K)),
                     constant_values=_PAD_MEAN)
    cov2 = cov.reshape(1, 1)

    out = pl.pallas_call(
        functools.partial(_body, K=K, B=B, D=D, nb=nb),
        grid=(nb,),
        in_specs=[
            pl.BlockSpec((B, D), lambda i: (0, 0)),
            pl.BlockSpec((D, _KB), lambda i: (0, i)),
            pl.BlockSpec((1, 1), lambda i: (0, 0)),
        ],
        out_specs=pl.BlockSpec((B, 1), lambda i: (0, 0)),
        out_shape=jax.ShapeDtypeStruct((B, 1), jnp.int32),
        scratch_shapes=[
            pltpu.VMEM((B, 1), jnp.float32),
            pltpu.VMEM((B, 1), jnp.int32),
        ],
    )(xs, meansT, cov2)
    return out[:, 0]
